# Initial kernel scaffold; baseline (speedup 1.0000x reference)
import jax
import jax.numpy as jnp
from jax.experimental import pallas as pl


def kernel(node_features, coord_features, indices, row_splits, W, b):
    raise NotImplementedError("write your pallas kernel here")


# speedup vs baseline: 16.8764x; 16.8764x over previous
import functools

import jax
import jax.numpy as jnp
from jax import lax
from jax.experimental import pallas as pl
from jax.experimental.pallas import tpu as pltpu
from jax.experimental.pallas import tpu_sc as plsc

NC = 2
NS = 16
NW = NC * NS
LANES = 16
CHUNK = 128
PCOLS = 256


def _dense_body(x_ref, w_ref, b_ref, o_ref):
    o_ref[...] = (
        jnp.dot(x_ref[...], w_ref[...], preferred_element_type=jnp.float32)
        + b_ref[...]
    )


def _dense(x, w, b):
    ni, fin = x.shape
    cols = w.shape[1]
    bm = 1024
    return pl.pallas_call(
        _dense_body,
        grid=(ni // bm,),
        in_specs=[
            pl.BlockSpec((bm, fin), lambda i: (i, 0)),
            pl.BlockSpec((fin, cols), lambda i: (0, 0)),
            pl.BlockSpec((1, cols), lambda i: (0, 0)),
        ],
        out_specs=pl.BlockSpec((bm, cols), lambda i: (i, 0)),
        out_shape=jax.ShapeDtypeStruct((ni, cols), jnp.float32),
    )(x, w, b)


def _sc_body(nf, crd, idx, rs, out, sl_v, idx_v, crd_v, rows_v, acc_v, sem,
             *, seg_w, units, d, ce):
    wid = lax.axis_index("s") * NC + lax.axis_index("c")
    s0 = pl.multiple_of(wid * seg_w, 8)
    pltpu.sync_copy(rs.at[pl.ds(s0, seg_w + LANES)], sl_v)
    e0 = sl_v[pl.ds(0, LANES)][0]
    e1 = sl_v[pl.ds(seg_w, LANES)][0]

    zf = jnp.zeros((LANES,), jnp.float32)

    def zero_row(r, carry):
        for k in range(units // LANES):
            acc_v[r, pl.ds(k * LANES, LANES)] = zf
        return carry

    lax.fori_loop(0, seg_w, zero_row, 0)

    ii = lax.iota(jnp.int32, LANES)
    base0 = jnp.bitwise_and(e0, jnp.int32(-8))
    nch = (e1 - base0 + jnp.int32(CHUNK - 1)) // jnp.int32(CHUNK)

    def chunk_body(kc, carry):
        base = pl.multiple_of(base0 + kc * CHUNK, 8)
        pltpu.sync_copy(idx.at[pl.ds(base, CHUNK)], idx_v)
        for dd in range(d):
            pltpu.sync_copy(crd.at[pl.ds(pl.multiple_of(dd * ce + base, 8),
                                         CHUNK)],
                            crd_v.at[pl.ds(dd * CHUNK, CHUNK)])
        pltpu.async_copy(nf.at[idx_v], rows_v, sem).wait()

        def group_body(g, gcarry):
            gb = g * LANES
            evec = base + gb + ii
            mask = (evec >= e0) & (evec < e1)
            lo = jnp.zeros((LANES,), jnp.int32)
            step = seg_w // 2
            while step >= 1:
                vals = plsc.load_gather(sl_v, [lo + step])
                lo = jnp.where(vals <= evec, lo + step, lo)
                step //= 2
            cvs = [crd_v[pl.ds(dd * CHUNK + gb, LANES)] for dd in range(d)]
            j16 = gb + ii

            def ublk(t, ucarry):
                for r in range(8):
                    u = t * 8 + r
                    ucol = jnp.zeros((LANES,), jnp.int32) + u
                    f = None
                    for dd in range(d):
                        v = plsc.load_gather(rows_v, [j16, ucol + dd * units])
                        fv = v * cvs[dd]
                        f = fv if f is None else f + fv
                    f = jnp.maximum(f, 0.0)
                    plsc.addupdate_scatter(acc_v, [lo, ucol], f, mask=mask)
                return ucarry

            lax.fori_loop(0, units // 8, ublk, 0)
            return gcarry

        lax.fori_loop(0, CHUNK // LANES, group_body, 0)
        return carry

    lax.fori_loop(0, nch, chunk_body, 0)
    pltpu.sync_copy(acc_v, out.at[pl.ds(s0, seg_w)])


def kernel(node_features, coord_features, indices, row_splits, W, b):
    ni, fin = node_features.shape
    e, d = coord_features.shape
    no = row_splits.shape[0] - 1
    units = W.shape[1] // d
    seg_w = no // NW

    wp = (W.astype(jnp.float32)
          .reshape(fin, units, d).transpose(0, 2, 1).reshape(fin, units * d))
    wp = jnp.pad(wp, ((0, 0), (0, PCOLS - units * d)))
    bp = (b.astype(jnp.float32)
          .reshape(units, d).transpose(1, 0).reshape(1, units * d))
    bp = jnp.pad(bp, ((0, 0), (0, PCOLS - units * d)))
    nf = _dense(node_features.astype(jnp.float32), wp, bp)

    ce = e + CHUNK
    crd_flat = jnp.pad(coord_features.astype(jnp.float32).T,
                       ((0, 0), (0, CHUNK))).reshape(-1)
    idx_pad = jnp.pad(indices.astype(jnp.int32), (0, CHUNK))
    rs_pad = jnp.concatenate([
        row_splits.astype(jnp.int32),
        jnp.full((LANES - 1,), jnp.int32(e)),
    ])

    mesh = plsc.VectorSubcoreMesh(core_axis_name="c", subcore_axis_name="s")
    sck = pl.kernel(
        functools.partial(_sc_body, seg_w=seg_w, units=units, d=d, ce=ce),
        out_type=jax.ShapeDtypeStruct((no, units), jnp.float32),
        mesh=mesh,
        scratch_types=[
            pltpu.VMEM((seg_w + LANES,), jnp.int32),
            pltpu.VMEM((CHUNK,), jnp.int32),
            pltpu.VMEM((d * CHUNK,), jnp.float32),
            pltpu.VMEM((CHUNK, PCOLS), jnp.float32),
            pltpu.VMEM((seg_w, units), jnp.float32),
            pltpu.SemaphoreType.DMA,
        ],
        compiler_params=pltpu.CompilerParams(needs_layout_passes=False,
                                             use_tc_tiling_on_sc=False),
    )
    return sck(nf, crd_flat, idx_pad, rs_pad)


# speedup vs baseline: 48.8688x; 2.8957x over previous
import functools

import jax
import jax.numpy as jnp
from jax import lax
from jax.experimental import pallas as pl
from jax.experimental.pallas import tpu as pltpu
from jax.experimental.pallas import tpu_sc as plsc

NC = 2
NS = 16
NW = NC * NS
LANES = 16
CHUNK = 128
PCOLS = 256


def _dense_body(x_ref, w_ref, b_ref, o_ref):
    o_ref[...] = (
        jnp.dot(x_ref[...], w_ref[...], preferred_element_type=jnp.float32)
        + b_ref[...]
    )


def _dense(x, w, b):
    ni, fin = x.shape
    cols = w.shape[1]
    bm = 1024
    return pl.pallas_call(
        _dense_body,
        grid=(ni // bm,),
        in_specs=[
            pl.BlockSpec((bm, fin), lambda i: (i, 0)),
            pl.BlockSpec((fin, cols), lambda i: (0, 0)),
            pl.BlockSpec((1, cols), lambda i: (0, 0)),
        ],
        out_specs=pl.BlockSpec((bm, cols), lambda i: (i, 0)),
        out_shape=jax.ShapeDtypeStruct((ni, cols), jnp.float32),
    )(x, w, b)


def _sc_body(nf, crd, idx, rs, out, sl_v, idx_v, crd_v, rows_v, acc_v, sem,
             *, seg_w, units, d, ce):
    wid = lax.axis_index("s") * NC + lax.axis_index("c")
    s0 = pl.multiple_of(wid * seg_w, 8)
    pltpu.sync_copy(rs.at[pl.ds(s0, seg_w + LANES)], sl_v)
    e0 = sl_v[pl.ds(0, LANES)][0]
    e1 = sl_v[pl.ds(seg_w, LANES)][0]

    zf = jnp.zeros((LANES,), jnp.float32)

    def zero_row(r, carry):
        for k in range(units // LANES):
            acc_v[r, pl.ds(k * LANES, LANES)] = zf
        return carry

    lax.fori_loop(0, seg_w, zero_row, 0)

    ii = lax.iota(jnp.int32, LANES)
    base0 = jnp.bitwise_and(e0, jnp.int32(-8))
    nch = (e1 - base0 + jnp.int32(CHUNK - 1)) // jnp.int32(CHUNK)

    def chunk_body(kc, carry):
        base = pl.multiple_of(base0 + kc * CHUNK, 8)
        pltpu.sync_copy(idx.at[pl.ds(base, CHUNK)], idx_v)
        for dd in range(d):
            pltpu.sync_copy(crd.at[pl.ds(pl.multiple_of(dd * ce + base, 8),
                                         CHUNK)],
                            crd_v.at[pl.ds(dd * CHUNK, CHUNK)])
        pltpu.async_copy(nf.at[idx_v], rows_v, sem).wait()

        def group_body(g, gcarry):
            gb = g * LANES
            evec = base + gb + ii
            mask = (evec >= e0) & (evec < e1)
            lo = jnp.zeros((LANES,), jnp.int32)
            step = seg_w // 2
            while step >= 1:
                vals = plsc.load_gather(sl_v, [lo + step])
                lo = jnp.where(vals <= evec, lo + step, lo)
                step //= 2
            cvs = [crd_v[pl.ds(dd * CHUNK + gb, LANES)] for dd in range(d)]
            j16 = gb + ii

            def ublk(t, ucarry):
                for r in range(LANES):
                    ucol = t * LANES + jnp.bitwise_and(ii + r, LANES - 1)
                    f = None
                    for dd in range(d):
                        v = plsc.load_gather(rows_v, [j16, ucol + dd * units])
                        fv = v * cvs[dd]
                        f = fv if f is None else f + fv
                    f = jnp.maximum(f, 0.0)
                    plsc.addupdate_scatter(acc_v, [lo, ucol], f, mask=mask)
                return ucarry

            lax.fori_loop(0, units // LANES, ublk, 0)
            return gcarry

        lax.fori_loop(0, CHUNK // LANES, group_body, 0)
        return carry

    lax.fori_loop(0, nch, chunk_body, 0)
    pltpu.sync_copy(acc_v, out.at[pl.ds(s0, seg_w)])


def kernel(node_features, coord_features, indices, row_splits, W, b):
    ni, fin = node_features.shape
    e, d = coord_features.shape
    no = row_splits.shape[0] - 1
    units = W.shape[1] // d
    seg_w = no // NW

    wp = (W.astype(jnp.float32)
          .reshape(fin, units, d).transpose(0, 2, 1).reshape(fin, units * d))
    wp = jnp.pad(wp, ((0, 0), (0, PCOLS - units * d)))
    bp = (b.astype(jnp.float32)
          .reshape(units, d).transpose(1, 0).reshape(1, units * d))
    bp = jnp.pad(bp, ((0, 0), (0, PCOLS - units * d)))
    nf = _dense(node_features.astype(jnp.float32), wp, bp)

    ce = e + CHUNK
    crd_flat = jnp.pad(coord_features.astype(jnp.float32).T,
                       ((0, 0), (0, CHUNK))).reshape(-1)
    idx_pad = jnp.pad(indices.astype(jnp.int32), (0, CHUNK))
    rs_pad = jnp.concatenate([
        row_splits.astype(jnp.int32),
        jnp.full((LANES - 1,), jnp.int32(e)),
    ])

    mesh = plsc.VectorSubcoreMesh(core_axis_name="c", subcore_axis_name="s")
    sck = pl.kernel(
        functools.partial(_sc_body, seg_w=seg_w, units=units, d=d, ce=ce),
        out_type=jax.ShapeDtypeStruct((no, units), jnp.float32),
        mesh=mesh,
        scratch_types=[
            pltpu.VMEM((seg_w + LANES,), jnp.int32),
            pltpu.VMEM((CHUNK,), jnp.int32),
            pltpu.VMEM((d * CHUNK,), jnp.float32),
            pltpu.VMEM((CHUNK, PCOLS), jnp.float32),
            pltpu.VMEM((seg_w, units), jnp.float32),
            pltpu.SemaphoreType.DMA,
        ],
        compiler_params=pltpu.CompilerParams(needs_layout_passes=False,
                                             use_tc_tiling_on_sc=False),
    )
    return sck(nf, crd_flat, idx_pad, rs_pad)
